# baseline (speedup 1.0000x reference)
import functools

import jax
import jax.numpy as jnp
from jax import lax
from jax.experimental import pallas as pl
from jax.experimental.pallas import tpu as pltpu
from jax.experimental.pallas import tpu_sc as plsc

N = 10000
D = 128
E = 320000
NC, NS = 2, 16
NW = NC * NS
CH = 128
K = 80
KC = 8
K0, K1 = 128, 32
EPAD = NW * K * CH
NROW = EPAD // CH
NSLICE = 632
NPAD = NS * NSLICE

_mesh = plsc.VectorSubcoreMesh(core_axis_name="c", subcore_axis_name="s")


@functools.partial(
    pl.kernel,
    out_type=(jax.ShapeDtypeStruct((NC, NPAD, D), jnp.float32),
              jax.ShapeDtypeStruct((NW, NPAD), jnp.float32)),
    mesh=_mesh,
    compiler_params=pltpu.CompilerParams(needs_layout_passes=False),
    scratch_types=[
        pltpu.VMEM((KC, CH), jnp.int32),
        pltpu.VMEM((KC, CH), jnp.int32),
        pltpu.VMEM((2, CH, D), jnp.float32),
        pltpu.VMEM((NPAD,), jnp.float32),
        pltpu.VMEM_SHARED((NPAD, D), jnp.float32),
        pltpu.SemaphoreType.DMA((2,)),
        pltpu.SemaphoreType.DMA((2,)),
    ],
)
def _sc_gather_segsum(feat, srcs, dsts, z_feat, z_hist,
                      agg_out, deg_out,
                      src_v, dst_v, rows_v, hist_v, agg_sh, gsem, ssem):
    cid = lax.axis_index("c")
    sid = lax.axis_index("s")
    wid = sid * NC + cid
    row0 = sid * NSLICE
    pltpu.sync_copy(z_feat, agg_sh.at[pl.ds(row0, NSLICE)])
    pltpu.sync_copy(z_hist, hist_v)
    plsc.subcore_barrier()

    ones16 = jnp.ones((16,), jnp.float32)

    def hist(j):
        for g in range(CH // 16):
            idx16 = dst_v[j, pl.ds(g * 16, 16)]
            plsc.addupdate_scatter(hist_v, [idx16], ones16)

    start = jnp.where(cid == 0, sid * K0, NS * K0 + sid * K1)
    nchunks = jnp.where(cid == 0, K0 // KC, K1 // KC)

    def chunk(c, carry):
        pltpu.sync_copy(srcs.at[pl.ds(start + c * KC, KC)], src_v)
        pltpu.sync_copy(dsts.at[pl.ds(start + c * KC, KC)], dst_v)
        gd = [
            pltpu.async_copy(feat.at[src_v.at[0]], rows_v.at[0], gsem.at[0]),
            pltpu.async_copy(feat.at[src_v.at[1]], rows_v.at[1], gsem.at[1]),
        ]
        sd = [None, None]
        for j in range(KC):
            b = j & 1
            gd[b].wait()
            sd[b] = pltpu.async_copy(rows_v.at[b], agg_sh.at[dst_v.at[j]],
                                     ssem.at[b], add=True)
            hist(j)
            if j + 2 < KC:
                sd[b].wait()
                gd[b] = pltpu.async_copy(feat.at[src_v.at[j + 2]],
                                         rows_v.at[b], gsem.at[b])
        sd[0].wait()
        sd[1].wait()
        return carry

    lax.fori_loop(0, nchunks, chunk, 0)
    plsc.subcore_barrier()
    pltpu.sync_copy(agg_sh.at[pl.ds(row0, NSLICE)],
                    agg_out.at[cid, pl.ds(row0, NSLICE)])
    pltpu.sync_copy(hist_v, deg_out.at[wid])


def _encode_body(x_ref, w_ref, o_ref):
    o_ref[...] = jnp.maximum(
        jnp.dot(x_ref[...], w_ref[...], preferred_element_type=jnp.float32),
        0.0)


def _encode(x, w):
    return pl.pallas_call(
        _encode_body,
        out_shape=jax.ShapeDtypeStruct((NPAD, D), jnp.float32),
    )(x, w)


def _combine_body(a0_ref, a1_ref, dp_ref, res_ref, w_ref, o_ref):
    agg = a0_ref[...] + a1_ref[...]
    deg = lax.dot_general(dp_ref[...], jnp.ones((NW, 1), jnp.float32),
                          (((0,), (0,)), ((), ())),
                          preferred_element_type=jnp.float32)
    mean = agg / jnp.maximum(deg, 1.0)
    o_ref[...] = jnp.maximum(
        jnp.dot(mean, w_ref[...], preferred_element_type=jnp.float32),
        0.0) + res_ref[...]


def _combine(agg_parts, deg_parts, res, w):
    return pl.pallas_call(
        _combine_body,
        out_shape=jax.ShapeDtypeStruct((NPAD, D), jnp.float32),
    )(agg_parts[0], agg_parts[1], deg_parts, res, w)


def _prep_edges(idx, fill):
    pad = jnp.full((EPAD - E,), fill, jnp.int32)
    return jnp.concatenate([idx, pad]).reshape(NROW, CH)


def kernel(d_feat, p_feat, W_d, W_p, W_mono, W_bip_dp, W_bip_pd,
           dd_edge_index, dp_edge_index):
    z_feat = jnp.zeros((NSLICE, D), jnp.float32)
    z_hist = jnp.zeros((NPAD,), jnp.float32)
    z_rows = jnp.zeros((NPAD - N, D), jnp.float32)

    d_att = _encode(jnp.concatenate([d_feat, z_rows]), W_d)
    p_att = _encode(jnp.concatenate([p_feat, z_rows]), W_p)

    dd_src = _prep_edges(dd_edge_index[0], 0)
    dd_dst = _prep_edges(dd_edge_index[1], N)
    dp_d2p_src = _prep_edges(dp_edge_index[0], 0)
    dp_d2p_dst = _prep_edges(dp_edge_index[1], N)
    dp_p2d_src = _prep_edges(dp_edge_index[1], 0)
    dp_p2d_dst = _prep_edges(dp_edge_index[0], N)

    a, g = _sc_gather_segsum(d_att, dd_src, dd_dst, z_feat, z_hist)
    d = _combine(a, g, d_att, W_mono)
    a, g = _sc_gather_segsum(d, dp_d2p_src, dp_d2p_dst, z_feat, z_hist)
    p = _combine(a, g, p_att, W_bip_dp)
    a, g = _sc_gather_segsum(p, dp_p2d_src, dp_p2d_dst, z_feat, z_hist)
    d = _combine(a, g, d, W_bip_pd)
    return jnp.stack([d[:N], p[:N], d_att[:N], p_att[:N]])


# speedup vs baseline: 1.0651x; 1.0651x over previous
import functools

import jax
import jax.numpy as jnp
from jax import lax
from jax.experimental import pallas as pl
from jax.experimental.pallas import tpu as pltpu
from jax.experimental.pallas import tpu_sc as plsc

N = 10000
D = 128
E = 320000
NC, NS = 2, 16
NW = NC * NS
CH = 128
K = 80
KC = 8
K0, K1 = 144, 16
EPAD = NW * K * CH
NROW = EPAD // CH
NSLICE = 632
NPAD = NS * NSLICE

_mesh = plsc.VectorSubcoreMesh(core_axis_name="c", subcore_axis_name="s")


@functools.partial(
    pl.kernel,
    out_type=(jax.ShapeDtypeStruct((NC, NPAD, D), jnp.float32),
              jax.ShapeDtypeStruct((NW, NPAD), jnp.float32)),
    mesh=_mesh,
    compiler_params=pltpu.CompilerParams(needs_layout_passes=False),
    scratch_types=[
        pltpu.VMEM((KC, CH), jnp.int32),
        pltpu.VMEM((KC, CH), jnp.int32),
        pltpu.VMEM((2, CH, D), jnp.float32),
        pltpu.VMEM((NPAD,), jnp.float32),
        pltpu.VMEM_SHARED((NPAD, D), jnp.float32),
        pltpu.SemaphoreType.DMA((2,)),
        pltpu.SemaphoreType.DMA((2,)),
    ],
)
def _sc_gather_segsum(feat, srcs, dsts, z_feat, z_hist,
                      agg_out, deg_out,
                      src_v, dst_v, rows_v, hist_v, agg_sh, gsem, ssem):
    cid = lax.axis_index("c")
    sid = lax.axis_index("s")
    wid = sid * NC + cid
    row0 = sid * NSLICE
    pltpu.sync_copy(z_feat, agg_sh.at[pl.ds(row0, NSLICE)])
    pltpu.sync_copy(z_hist, hist_v)
    plsc.subcore_barrier()

    ones16 = jnp.ones((16,), jnp.float32)

    def hist(j):
        for g in range(CH // 16):
            idx16 = dst_v[j, pl.ds(g * 16, 16)]
            plsc.addupdate_scatter(hist_v, [idx16], ones16)

    start = jnp.where(cid == 0, sid * K0, NS * K0 + sid * K1)
    nchunks = jnp.where(cid == 0, K0 // KC, K1 // KC)

    def chunk(c, carry):
        pltpu.sync_copy(srcs.at[pl.ds(start + c * KC, KC)], src_v)
        pltpu.sync_copy(dsts.at[pl.ds(start + c * KC, KC)], dst_v)
        gd = [
            pltpu.async_copy(feat.at[src_v.at[0]], rows_v.at[0], gsem.at[0]),
            pltpu.async_copy(feat.at[src_v.at[1]], rows_v.at[1], gsem.at[1]),
        ]
        sd = [None, None]
        for j in range(KC):
            b = j & 1
            gd[b].wait()
            sd[b] = pltpu.async_copy(rows_v.at[b], agg_sh.at[dst_v.at[j]],
                                     ssem.at[b], add=True)
            hist(j)
            if j + 2 < KC:
                sd[b].wait()
                gd[b] = pltpu.async_copy(feat.at[src_v.at[j + 2]],
                                         rows_v.at[b], gsem.at[b])
        sd[0].wait()
        sd[1].wait()
        return carry

    lax.fori_loop(0, nchunks, chunk, 0)
    plsc.subcore_barrier()
    pltpu.sync_copy(agg_sh.at[pl.ds(row0, NSLICE)],
                    agg_out.at[cid, pl.ds(row0, NSLICE)])
    pltpu.sync_copy(hist_v, deg_out.at[wid])


def _encode_body(x_ref, w_ref, o_ref):
    o_ref[...] = jnp.maximum(
        jnp.dot(x_ref[...], w_ref[...], preferred_element_type=jnp.float32),
        0.0)


def _encode(x, w):
    return pl.pallas_call(
        _encode_body,
        out_shape=jax.ShapeDtypeStruct((NPAD, D), jnp.float32),
    )(x, w)


def _combine_body(a0_ref, a1_ref, dp_ref, res_ref, w_ref, o_ref):
    agg = a0_ref[...] + a1_ref[...]
    deg = lax.dot_general(dp_ref[...], jnp.ones((NW, 1), jnp.float32),
                          (((0,), (0,)), ((), ())),
                          preferred_element_type=jnp.float32)
    mean = agg / jnp.maximum(deg, 1.0)
    o_ref[...] = jnp.maximum(
        jnp.dot(mean, w_ref[...], preferred_element_type=jnp.float32),
        0.0) + res_ref[...]


def _combine(agg_parts, deg_parts, res, w):
    return pl.pallas_call(
        _combine_body,
        out_shape=jax.ShapeDtypeStruct((NPAD, D), jnp.float32),
    )(agg_parts[0], agg_parts[1], deg_parts, res, w)


def _prep_edges(idx, fill):
    pad = jnp.full((EPAD - E,), fill, jnp.int32)
    return jnp.concatenate([idx, pad]).reshape(NROW, CH)


def kernel(d_feat, p_feat, W_d, W_p, W_mono, W_bip_dp, W_bip_pd,
           dd_edge_index, dp_edge_index):
    z_feat = jnp.zeros((NSLICE, D), jnp.float32)
    z_hist = jnp.zeros((NPAD,), jnp.float32)
    z_rows = jnp.zeros((NPAD - N, D), jnp.float32)

    d_att = _encode(jnp.concatenate([d_feat, z_rows]), W_d)
    p_att = _encode(jnp.concatenate([p_feat, z_rows]), W_p)

    dd_src = _prep_edges(dd_edge_index[0], 0)
    dd_dst = _prep_edges(dd_edge_index[1], N)
    dp_d2p_src = _prep_edges(dp_edge_index[0], 0)
    dp_d2p_dst = _prep_edges(dp_edge_index[1], N)
    dp_p2d_src = _prep_edges(dp_edge_index[1], 0)
    dp_p2d_dst = _prep_edges(dp_edge_index[0], N)

    a, g = _sc_gather_segsum(d_att, dd_src, dd_dst, z_feat, z_hist)
    d = _combine(a, g, d_att, W_mono)
    a, g = _sc_gather_segsum(d, dp_d2p_src, dp_d2p_dst, z_feat, z_hist)
    p = _combine(a, g, p_att, W_bip_dp)
    a, g = _sc_gather_segsum(p, dp_p2d_src, dp_p2d_dst, z_feat, z_hist)
    d = _combine(a, g, d, W_bip_pd)
    return jnp.stack([d[:N], p[:N], d_att[:N], p_att[:N]])


# speedup vs baseline: 1.0700x; 1.0045x over previous
import functools

import jax
import jax.numpy as jnp
from jax import lax
from jax.experimental import pallas as pl
from jax.experimental.pallas import tpu as pltpu
from jax.experimental.pallas import tpu_sc as plsc

N = 10000
D = 128
E = 320000
NC, NS = 2, 16
NW = NC * NS
CH = 128
K = 80
KC = 8
K0, K1 = 152, 8
EPAD = NW * K * CH
NROW = EPAD // CH
NSLICE = 632
NPAD = NS * NSLICE

_mesh = plsc.VectorSubcoreMesh(core_axis_name="c", subcore_axis_name="s")


@functools.partial(
    pl.kernel,
    out_type=(jax.ShapeDtypeStruct((NC, NPAD, D), jnp.float32),
              jax.ShapeDtypeStruct((NW, NPAD), jnp.float32)),
    mesh=_mesh,
    compiler_params=pltpu.CompilerParams(needs_layout_passes=False),
    scratch_types=[
        pltpu.VMEM((KC, CH), jnp.int32),
        pltpu.VMEM((KC, CH), jnp.int32),
        pltpu.VMEM((2, CH, D), jnp.float32),
        pltpu.VMEM((NPAD,), jnp.float32),
        pltpu.VMEM_SHARED((NPAD, D), jnp.float32),
        pltpu.SemaphoreType.DMA((2,)),
        pltpu.SemaphoreType.DMA((2,)),
    ],
)
def _sc_gather_segsum(feat, srcs, dsts, z_feat, z_hist,
                      agg_out, deg_out,
                      src_v, dst_v, rows_v, hist_v, agg_sh, gsem, ssem):
    cid = lax.axis_index("c")
    sid = lax.axis_index("s")
    wid = sid * NC + cid
    row0 = sid * NSLICE
    pltpu.sync_copy(z_feat, agg_sh.at[pl.ds(row0, NSLICE)])
    pltpu.sync_copy(z_hist, hist_v)
    plsc.subcore_barrier()

    ones16 = jnp.ones((16,), jnp.float32)

    def hist(j):
        for g in range(CH // 16):
            idx16 = dst_v[j, pl.ds(g * 16, 16)]
            plsc.addupdate_scatter(hist_v, [idx16], ones16)

    start = jnp.where(cid == 0, sid * K0, NS * K0 + sid * K1)
    nchunks = jnp.where(cid == 0, K0 // KC, K1 // KC)

    def chunk(c, carry):
        pltpu.sync_copy(srcs.at[pl.ds(start + c * KC, KC)], src_v)
        pltpu.sync_copy(dsts.at[pl.ds(start + c * KC, KC)], dst_v)
        gd = [
            pltpu.async_copy(feat.at[src_v.at[0]], rows_v.at[0], gsem.at[0]),
            pltpu.async_copy(feat.at[src_v.at[1]], rows_v.at[1], gsem.at[1]),
        ]
        sd = [None, None]
        for j in range(KC):
            b = j & 1
            gd[b].wait()
            sd[b] = pltpu.async_copy(rows_v.at[b], agg_sh.at[dst_v.at[j]],
                                     ssem.at[b], add=True)
            hist(j)
            if j + 2 < KC:
                sd[b].wait()
                gd[b] = pltpu.async_copy(feat.at[src_v.at[j + 2]],
                                         rows_v.at[b], gsem.at[b])
        sd[0].wait()
        sd[1].wait()
        return carry

    lax.fori_loop(0, nchunks, chunk, 0)
    plsc.subcore_barrier()
    pltpu.sync_copy(agg_sh.at[pl.ds(row0, NSLICE)],
                    agg_out.at[cid, pl.ds(row0, NSLICE)])
    pltpu.sync_copy(hist_v, deg_out.at[wid])


def _encode_body(x_ref, w_ref, o_ref):
    o_ref[...] = jnp.maximum(
        jnp.dot(x_ref[...], w_ref[...], preferred_element_type=jnp.float32),
        0.0)


def _encode(x, w):
    return pl.pallas_call(
        _encode_body,
        out_shape=jax.ShapeDtypeStruct((NPAD, D), jnp.float32),
    )(x, w)


def _combine_body(a0_ref, a1_ref, dp_ref, res_ref, w_ref, o_ref):
    agg = a0_ref[...] + a1_ref[...]
    deg = lax.dot_general(dp_ref[...], jnp.ones((NW, 1), jnp.float32),
                          (((0,), (0,)), ((), ())),
                          preferred_element_type=jnp.float32)
    mean = agg / jnp.maximum(deg, 1.0)
    o_ref[...] = jnp.maximum(
        jnp.dot(mean, w_ref[...], preferred_element_type=jnp.float32),
        0.0) + res_ref[...]


def _combine(agg_parts, deg_parts, res, w):
    return pl.pallas_call(
        _combine_body,
        out_shape=jax.ShapeDtypeStruct((NPAD, D), jnp.float32),
    )(agg_parts[0], agg_parts[1], deg_parts, res, w)


def _prep_edges(idx, fill):
    pad = jnp.full((EPAD - E,), fill, jnp.int32)
    return jnp.concatenate([idx, pad]).reshape(NROW, CH)


def kernel(d_feat, p_feat, W_d, W_p, W_mono, W_bip_dp, W_bip_pd,
           dd_edge_index, dp_edge_index):
    z_feat = jnp.zeros((NSLICE, D), jnp.float32)
    z_hist = jnp.zeros((NPAD,), jnp.float32)
    z_rows = jnp.zeros((NPAD - N, D), jnp.float32)

    d_att = _encode(jnp.concatenate([d_feat, z_rows]), W_d)
    p_att = _encode(jnp.concatenate([p_feat, z_rows]), W_p)

    dd_src = _prep_edges(dd_edge_index[0], 0)
    dd_dst = _prep_edges(dd_edge_index[1], N)
    dp_d2p_src = _prep_edges(dp_edge_index[0], 0)
    dp_d2p_dst = _prep_edges(dp_edge_index[1], N)
    dp_p2d_src = _prep_edges(dp_edge_index[1], 0)
    dp_p2d_dst = _prep_edges(dp_edge_index[0], N)

    a, g = _sc_gather_segsum(d_att, dd_src, dd_dst, z_feat, z_hist)
    d = _combine(a, g, d_att, W_mono)
    a, g = _sc_gather_segsum(d, dp_d2p_src, dp_d2p_dst, z_feat, z_hist)
    p = _combine(a, g, p_att, W_bip_dp)
    a, g = _sc_gather_segsum(p, dp_p2d_src, dp_p2d_dst, z_feat, z_hist)
    d = _combine(a, g, d, W_bip_pd)
    return jnp.stack([d[:N], p[:N], d_att[:N], p_att[:N]])


# speedup vs baseline: 1.0722x; 1.0021x over previous
import functools

import jax
import jax.numpy as jnp
from jax import lax
from jax.experimental import pallas as pl
from jax.experimental.pallas import tpu as pltpu
from jax.experimental.pallas import tpu_sc as plsc

N = 10000
D = 128
E = 320000
NC, NS = 2, 16
NW = NC * NS
CH = 128
K = 80
KC = 8
K0, K1 = 152, 8
EPAD = NW * K * CH
NROW = EPAD // CH
NSLICE = 632
NPAD = NS * NSLICE

_mesh = plsc.VectorSubcoreMesh(core_axis_name="c", subcore_axis_name="s")


@functools.partial(
    pl.kernel,
    out_type=(jax.ShapeDtypeStruct((NC, NPAD, D), jnp.float32),
              jax.ShapeDtypeStruct((NW, NPAD), jnp.float32)),
    mesh=_mesh,
    compiler_params=pltpu.CompilerParams(needs_layout_passes=False),
    scratch_types=[
        pltpu.VMEM((2, KC, CH), jnp.int32),
        pltpu.VMEM((2, KC, CH), jnp.int32),
        pltpu.VMEM((2, CH, D), jnp.float32),
        pltpu.VMEM((NPAD,), jnp.float32),
        pltpu.VMEM_SHARED((NPAD, D), jnp.float32),
        pltpu.SemaphoreType.DMA((2,)),
        pltpu.SemaphoreType.DMA((2,)),
        pltpu.SemaphoreType.DMA,
    ],
)
def _sc_gather_segsum(feat, srcs, dsts, z_feat, z_hist,
                      agg_out, deg_out,
                      src_v, dst_v, rows_v, hist_v, agg_sh, gsem, ssem, isem):
    cid = lax.axis_index("c")
    sid = lax.axis_index("s")
    wid = sid * NC + cid
    row0 = sid * NSLICE
    pltpu.sync_copy(z_feat, agg_sh.at[pl.ds(row0, NSLICE)])
    pltpu.sync_copy(z_hist, hist_v)
    plsc.subcore_barrier()

    ones16 = jnp.ones((16,), jnp.float32)

    start = jnp.where(cid == 0, sid * K0, NS * K0 + sid * K1)
    nchunks = jnp.where(cid == 0, K0 // KC, K1 // KC)

    def wait_idx_pair():
        pltpu.make_async_copy(srcs.at[pl.ds(0, KC)], src_v.at[0], isem).wait()
        pltpu.make_async_copy(dsts.at[pl.ds(0, KC)], dst_v.at[0], isem).wait()

    pltpu.async_copy(srcs.at[pl.ds(start, KC)], src_v.at[0], isem)
    pltpu.async_copy(dsts.at[pl.ds(start, KC)], dst_v.at[0], isem)
    wait_idx_pair()
    pltpu.async_copy(feat.at[src_v.at[0, 0]], rows_v.at[0], gsem.at[0])
    pltpu.async_copy(feat.at[src_v.at[0, 1]], rows_v.at[1], gsem.at[1])

    def chunk(c, carry):
        p = c % 2
        np_ = (c + 1) % 2
        more = c + 1 < nchunks

        @pl.when(more)
        def _():
            nx = start + (c + 1) * KC
            pltpu.async_copy(srcs.at[pl.ds(nx, KC)], src_v.at[np_], isem)
            pltpu.async_copy(dsts.at[pl.ds(nx, KC)], dst_v.at[np_], isem)

        for j in range(KC):
            b = j & 1
            pltpu.make_async_copy(feat.at[pl.ds(0, CH)], rows_v.at[b],
                                  gsem.at[b]).wait()
            pltpu.async_copy(rows_v.at[b], agg_sh.at[dst_v.at[p, j]],
                             ssem.at[b], add=True)
            for g in range(CH // 16):
                idx16 = dst_v[p, j, pl.ds(g * 16, 16)]
                plsc.addupdate_scatter(hist_v, [idx16], ones16)
            if j == KC - 2:
                @pl.when(more)
                def _():
                    wait_idx_pair()
            pltpu.make_async_copy(rows_v.at[b], agg_sh.at[pl.ds(0, CH)],
                                  ssem.at[b]).wait()
            if j + 2 < KC:
                pltpu.async_copy(feat.at[src_v.at[p, j + 2]], rows_v.at[b],
                                 gsem.at[b])
            else:
                @pl.when(more)
                def _():
                    pltpu.async_copy(feat.at[src_v.at[np_, j + 2 - KC]],
                                     rows_v.at[b], gsem.at[b])
        return carry

    lax.fori_loop(0, nchunks, chunk, 0)
    plsc.subcore_barrier()
    pltpu.sync_copy(agg_sh.at[pl.ds(row0, NSLICE)],
                    agg_out.at[cid, pl.ds(row0, NSLICE)])
    pltpu.sync_copy(hist_v, deg_out.at[wid])


def _encode_body(x_ref, w_ref, o_ref):
    o_ref[...] = jnp.maximum(
        jnp.dot(x_ref[...], w_ref[...], preferred_element_type=jnp.float32),
        0.0)


def _encode(x, w):
    return pl.pallas_call(
        _encode_body,
        out_shape=jax.ShapeDtypeStruct((NPAD, D), jnp.float32),
    )(x, w)


def _combine_body(a0_ref, a1_ref, dp_ref, res_ref, w_ref, o_ref):
    agg = a0_ref[...] + a1_ref[...]
    deg = lax.dot_general(dp_ref[...], jnp.ones((NW, 1), jnp.float32),
                          (((0,), (0,)), ((), ())),
                          preferred_element_type=jnp.float32)
    mean = agg / jnp.maximum(deg, 1.0)
    o_ref[...] = jnp.maximum(
        jnp.dot(mean, w_ref[...], preferred_element_type=jnp.float32),
        0.0) + res_ref[...]


def _combine(agg_parts, deg_parts, res, w):
    return pl.pallas_call(
        _combine_body,
        out_shape=jax.ShapeDtypeStruct((NPAD, D), jnp.float32),
    )(agg_parts[0], agg_parts[1], deg_parts, res, w)


def _prep_edges(idx, fill):
    pad = jnp.full((EPAD - E,), fill, jnp.int32)
    return jnp.concatenate([idx, pad]).reshape(NROW, CH)


def kernel(d_feat, p_feat, W_d, W_p, W_mono, W_bip_dp, W_bip_pd,
           dd_edge_index, dp_edge_index):
    z_feat = jnp.zeros((NSLICE, D), jnp.float32)
    z_hist = jnp.zeros((NPAD,), jnp.float32)
    z_rows = jnp.zeros((NPAD - N, D), jnp.float32)

    d_att = _encode(jnp.concatenate([d_feat, z_rows]), W_d)
    p_att = _encode(jnp.concatenate([p_feat, z_rows]), W_p)

    dd_src = _prep_edges(dd_edge_index[0], 0)
    dd_dst = _prep_edges(dd_edge_index[1], N)
    dp_d2p_src = _prep_edges(dp_edge_index[0], 0)
    dp_d2p_dst = _prep_edges(dp_edge_index[1], N)
    dp_p2d_src = _prep_edges(dp_edge_index[1], 0)
    dp_p2d_dst = _prep_edges(dp_edge_index[0], N)

    a, g = _sc_gather_segsum(d_att, dd_src, dd_dst, z_feat, z_hist)
    d = _combine(a, g, d_att, W_mono)
    a, g = _sc_gather_segsum(d, dp_d2p_src, dp_d2p_dst, z_feat, z_hist)
    p = _combine(a, g, p_att, W_bip_dp)
    a, g = _sc_gather_segsum(p, dp_p2d_src, dp_p2d_dst, z_feat, z_hist)
    d = _combine(a, g, d, W_bip_pd)
    return jnp.stack([d[:N], p[:N], d_att[:N], p_att[:N]])
